# baseline (speedup 1.0000x reference)
import jax
import jax.numpy as jnp
from jax import lax
from jax.experimental import pallas as pl
from jax.experimental.pallas import tpu as pltpu
from jax.experimental.pallas import tpu_sc as plsc

N = 10000
E = 160000
IN = 512
HID = 512
OUT = 256
K = 10
FQ = OUT // 4

NTILES = 16
NPAD = 10240
ROWS_PROC = 640
NTAIL = N - 15 * ROWS_PROC
EDGES_PER_TILE = E // NTILES
ECHUNK = 128
NCHUNKS = 79
EPT_PAD = NCHUNKS * ECHUNK
NBUF = 3


def _mlp_body(x_ref, w1_ref, b1_ref, w2_ref, b2_ref, out_ref):
    h1 = jnp.maximum(
        jnp.dot(x_ref[...], w1_ref[...], preferred_element_type=jnp.float32)
        + b1_ref[...], 0.0)
    h2 = (jnp.dot(h1, w2_ref[...], preferred_element_type=jnp.float32)
          + b2_ref[...])
    for q in range(4):
        out_ref[q] = h2[:, q * FQ:(q + 1) * FQ]


def _mlp(x, W1, b1, W2, b2):
    BM = 400
    grid = (N // BM,)
    return pl.pallas_call(
        _mlp_body,
        grid=grid,
        in_specs=[
            pl.BlockSpec((BM, IN), lambda i: (i, 0)),
            pl.BlockSpec((IN, HID), lambda i: (0, 0)),
            pl.BlockSpec((1, HID), lambda i: (0, 0)),
            pl.BlockSpec((HID, OUT), lambda i: (0, 0)),
            pl.BlockSpec((1, OUT), lambda i: (0, 0)),
        ],
        out_specs=pl.BlockSpec((4, BM, FQ), lambda i: (0, i, 0)),
        out_shape=jax.ShapeDtypeStruct((4, N, FQ), jnp.float32),
    )(x, W1, b1.reshape(1, HID), W2, b2.reshape(1, OUT))


def _rsqrt16(d):
    i = plsc.bitcast(d, jnp.int32)
    i = jnp.int32(0x5F3759DF) - lax.shift_right_arithmetic(i, 1)
    y = plsc.bitcast(i, jnp.float32)
    for _ in range(4):
        y = y * (1.5 - 0.5 * d * y * y)
    return y


def _deg_body(dst_ref, dinv_ref, part, dstb, parts_sh, sumb, dinvb, sem):
    sid = lax.axis_index("s")
    base = sid * ROWS_PROC
    zero16 = jnp.zeros((16,), jnp.float32)
    def _z(i, _):
        part[pl.ds(i * 16, 16)] = zero16
        return 0
    lax.fori_loop(0, NPAD // 16, _z, 0)
    pltpu.sync_copy(dst_ref.at[sid], dstb)
    ones16 = jnp.ones((16,), jnp.float32)
    def _scat(j, _):
        def _inner(kk, _):
            idx = dstb[j, 0, pl.ds(kk * 16, 16)]
            plsc.addupdate_scatter(part, [idx], ones16)
            return 0
        lax.fori_loop(0, ECHUNK // 16, _inner, 0)
        return 0
    lax.fori_loop(0, NCHUNKS, _scat, 0)
    pltpu.sync_copy(part, parts_sh.at[sid, 0])
    plsc.subcore_barrier()
    pltpu.sync_copy(parts_sh.at[:, :, pl.ds(base, ROWS_PROC)], sumb)
    def _red(c, _):
        acc = jnp.ones((16,), jnp.float32)
        for p in range(NTILES):
            acc = acc + sumb[p, 0, pl.ds(c * 16, 16)]
        dinvb[pl.ds(c * 16, 16)] = _rsqrt16(acc)
        return 0
    lax.fori_loop(0, ROWS_PROC // 16, _red, 0)
    pltpu.sync_copy(dinvb, dinv_ref.at[pl.ds(base, ROWS_PROC)])


def _degree_dinv(dst4d):
    mesh = plsc.VectorSubcoreMesh(core_axis_name="c", subcore_axis_name="s")
    f = pl.kernel(
        _deg_body,
        out_type=jax.ShapeDtypeStruct((NPAD,), jnp.float32),
        mesh=mesh,
        compiler_params=pltpu.CompilerParams(needs_layout_passes=False, use_tc_tiling_on_sc=False),
        scratch_types=[
            pltpu.VMEM((NPAD,), jnp.float32),
            pltpu.VMEM((NCHUNKS, 1, ECHUNK), jnp.int32),
            pltpu.VMEM_SHARED((NTILES, 1, NPAD), jnp.float32),
            pltpu.VMEM((NTILES, 1, ROWS_PROC), jnp.float32),
            pltpu.VMEM((ROWS_PROC,), jnp.float32),
            pltpu.SemaphoreType.DMA,
        ],
    )
    return f(dst4d)


def _scale_rows(rowb, vecb):
    def _row(r, _):
        dv = plsc.load_gather(vecb, [jnp.full((16,), r, jnp.int32)])
        for j in range(FQ // 16):
            sl = pl.ds(j * 16, 16)
            rowb[r, sl] = rowb[r, sl] * dv
        return 0
    lax.fori_loop(0, ROWS_PROC, _row, 0)


def _hop_body(h_ref, dinv_ref, src_ref, dst_ref, u_ref,
              acc_sh, rowb, gbuf, srcb, dstb, dinvb, d2b, sem, ssem):
    cid = lax.axis_index("c")
    sid = lax.axis_index("s")
    base = sid * ROWS_PROC
    slab = pl.ds(base, ROWS_PROC)

    pltpu.sync_copy(src_ref.at[sid], srcb)
    pltpu.sync_copy(dst_ref.at[sid], dstb)
    pltpu.sync_copy(dinv_ref.at[slab], dinvb)
    def _sq(i, _):
        dv = dinvb[pl.ds(i * 16, 16)]
        d2b[pl.ds(i * 16, 16)] = dv * dv
        return 0
    lax.fori_loop(0, ROWS_PROC // 16, _sq, 0)

    for q in range(2):
        qq = 2 * cid + q
        @pl.when(sid < NTILES - 1)
        def _():
            pltpu.sync_copy(h_ref.at[qq, slab], rowb)
        @pl.when(sid == NTILES - 1)
        def _():
            pltpu.sync_copy(h_ref.at[qq, pl.ds(N - NTAIL, NTAIL)],
                            rowb.at[pl.ds(0, NTAIL)])
            zero16 = jnp.zeros((16,), jnp.float32)
            def _zp(r, _):
                for j in range(FQ // 16):
                    rowb[NTAIL + r, pl.ds(j * 16, 16)] = zero16
                return 0
            lax.fori_loop(0, ROWS_PROC - NTAIL, _zp, 0)
        _scale_rows(rowb, dinvb)
        pltpu.sync_copy(rowb, u_ref.at[qq, 0, slab])
        pltpu.sync_copy(rowb, acc_sh.at[slab])
        plsc.subcore_barrier()

        def _hop(k, _):
            src_view = u_ref.at[qq, k]
            for p in range(NBUF - 1):
                pltpu.async_copy(src_view.at[srcb.at[p, 0]], gbuf.at[p], sem)
            def _edge(j, _):
                b = lax.rem(j, NBUF)
                pltpu.make_async_copy(
                    src_view.at[srcb.at[j, 0]], gbuf.at[b], sem).wait()
                @pl.when(j >= 1)
                def _():
                    bp = lax.rem(j + NBUF - 1, NBUF)
                    pltpu.make_async_copy(
                        gbuf.at[bp], acc_sh.at[dstb.at[j - 1, 0]], ssem).wait()
                @pl.when(j < NCHUNKS - (NBUF - 1))
                def _():
                    pltpu.async_copy(
                        src_view.at[srcb.at[j + NBUF - 1, 0]],
                        gbuf.at[lax.rem(j + NBUF - 1, NBUF)], sem)
                pltpu.async_copy(gbuf.at[b], acc_sh.at[dstb.at[j, 0]], ssem,
                                 add=True)
                return 0
            lax.fori_loop(0, NCHUNKS, _edge, 0)
            pltpu.make_async_copy(
                gbuf.at[lax.rem(NCHUNKS - 1, NBUF)],
                acc_sh.at[dstb.at[NCHUNKS - 1, 0]], ssem).wait()
            plsc.subcore_barrier()
            pltpu.sync_copy(acc_sh.at[slab], rowb)
            _scale_rows(rowb, d2b)
            pltpu.sync_copy(rowb, u_ref.at[qq, k + 1, slab])
            @pl.when(k < K - 1)
            def _():
                pltpu.sync_copy(rowb, acc_sh.at[slab])
            plsc.subcore_barrier()
            return 0
        lax.fori_loop(0, K, _hop, 0)


def _khop(h_quarters, dinv, src4d, dst4d):
    mesh = plsc.VectorSubcoreMesh(core_axis_name="c", subcore_axis_name="s")
    f = pl.kernel(
        _hop_body,
        out_type=jax.ShapeDtypeStruct((4, K + 1, NPAD, FQ), jnp.float32),
        mesh=mesh,
        compiler_params=pltpu.CompilerParams(needs_layout_passes=False, use_tc_tiling_on_sc=False),
        scratch_types=[
            pltpu.VMEM_SHARED((NPAD, FQ), jnp.float32),
            pltpu.VMEM((ROWS_PROC, FQ), jnp.float32),
            pltpu.VMEM((NBUF, ECHUNK, FQ), jnp.float32),
            pltpu.VMEM((NCHUNKS, 1, ECHUNK), jnp.int32),
            pltpu.VMEM((NCHUNKS, 1, ECHUNK), jnp.int32),
            pltpu.VMEM((ROWS_PROC,), jnp.float32),
            pltpu.VMEM((ROWS_PROC,), jnp.float32),
            pltpu.SemaphoreType.DMA,
            pltpu.SemaphoreType.DMA,
        ],
    )
    return f(h_quarters, dinv, src4d, dst4d)


def _combine_body(h_ref, u_refq, dinv_ref, pw_ref, pb_ref, out_ref):
    pw = pw_ref[...]
    pb = pb_ref[0, 0]
    d12 = 1.0 / dinv_ref[...]
    hh = h_ref[...]
    uu = u_refq[...]
    acc = jnp.zeros_like(out_ref)
    for k in range(K + 1):
        if k == 0:
            pk = jnp.concatenate([hh[q] for q in range(4)], axis=-1)
        else:
            uk = jnp.concatenate([uu[q, k] for q in range(4)], axis=-1)
            pk = uk * d12
        s = jax.nn.sigmoid(jnp.sum(pk * pw, axis=1) + pb)
        acc = acc + s[:, None] * pk
    out_ref[...] = acc


def _combine(h_quarters, u_all, dinv, proj_w, proj_b):
    BN = 400
    grid = (N // BN,)
    return pl.pallas_call(
        _combine_body,
        grid=grid,
        in_specs=[
            pl.BlockSpec((4, BN, FQ), lambda i: (0, i, 0)),
            pl.BlockSpec((4, K + 1, BN, FQ), lambda i: (0, 0, i, 0)),
            pl.BlockSpec((BN, 1), lambda i: (i, 0)),
            pl.BlockSpec((1, OUT), lambda i: (0, 0)),
            pl.BlockSpec((1, 1), lambda i: (0, 0)),
        ],
        out_specs=pl.BlockSpec((BN, OUT), lambda i: (i, 0)),
        out_shape=jax.ShapeDtypeStruct((N, OUT), jnp.float32),
    )(h_quarters, u_all, dinv.reshape(NPAD, 1), proj_w.reshape(1, OUT),
      jnp.asarray(proj_b, jnp.float32).reshape(1, 1))


@jax.jit
def kernel(x, edge_index, W1, b1, W2, b2, proj_w, proj_b):
    pad = jnp.full((2, NTILES, EPT_PAD - EDGES_PER_TILE), NPAD - 1, jnp.int32)
    ei = jnp.concatenate(
        [edge_index.reshape(2, NTILES, EDGES_PER_TILE), pad], axis=2)
    src4d = ei[0].reshape(NTILES, NCHUNKS, 1, ECHUNK)
    dst4d = ei[1].reshape(NTILES, NCHUNKS, 1, ECHUNK)
    h_quarters = _mlp(x, W1, b1, W2, b2)
    dinv = _degree_dinv(dst4d)
    u_all = _khop(h_quarters, dinv, src4d, dst4d)
    return _combine(h_quarters, u_all, dinv, proj_w, proj_b)


# speedup vs baseline: 1.0265x; 1.0265x over previous
import jax
import jax.numpy as jnp
from jax import lax
from jax.experimental import pallas as pl
from jax.experimental.pallas import tpu as pltpu
from jax.experimental.pallas import tpu_sc as plsc

N = 10000
E = 160000
IN = 512
HID = 512
OUT = 256
K = 10
FQ = OUT // 4

NTILES = 16
NPAD = 10240
ROWS_PROC = 640
NTAIL = N - 15 * ROWS_PROC
EDGES_PER_TILE = E // NTILES
ECHUNK = 128
NCHUNKS = 79
EPT_PAD = NCHUNKS * ECHUNK
NBUF = 3


def _mlp_body(x_ref, w1_ref, b1_ref, w2_ref, b2_ref, out_ref):
    h1 = jnp.maximum(
        jnp.dot(x_ref[...], w1_ref[...], preferred_element_type=jnp.float32)
        + b1_ref[...], 0.0)
    h2 = (jnp.dot(h1, w2_ref[...], preferred_element_type=jnp.float32)
          + b2_ref[...])
    for q in range(4):
        out_ref[q] = h2[:, q * FQ:(q + 1) * FQ]


def _mlp(x, W1, b1, W2, b2):
    BM = 400
    grid = (N // BM,)
    return pl.pallas_call(
        _mlp_body,
        grid=grid,
        in_specs=[
            pl.BlockSpec((BM, IN), lambda i: (i, 0)),
            pl.BlockSpec((IN, HID), lambda i: (0, 0)),
            pl.BlockSpec((1, HID), lambda i: (0, 0)),
            pl.BlockSpec((HID, OUT), lambda i: (0, 0)),
            pl.BlockSpec((1, OUT), lambda i: (0, 0)),
        ],
        out_specs=pl.BlockSpec((4, BM, FQ), lambda i: (0, i, 0)),
        out_shape=jax.ShapeDtypeStruct((4, N, FQ), jnp.float32),
    )(x, W1, b1.reshape(1, HID), W2, b2.reshape(1, OUT))


def _rsqrt16(d):
    i = plsc.bitcast(d, jnp.int32)
    i = jnp.int32(0x5F3759DF) - lax.shift_right_arithmetic(i, 1)
    y = plsc.bitcast(i, jnp.float32)
    for _ in range(4):
        y = y * (1.5 - 0.5 * d * y * y)
    return y


def _deg_body(dst_ref, dinv_ref, part, dstb, parts_sh, sumb, dinvb, sem):
    sid = lax.axis_index("s")
    base = sid * ROWS_PROC
    zero16 = jnp.zeros((16,), jnp.float32)
    def _z(i, _):
        part[pl.ds(i * 16, 16)] = zero16
        return 0
    lax.fori_loop(0, NPAD // 16, _z, 0)
    pltpu.sync_copy(dst_ref.at[sid], dstb)
    ones16 = jnp.ones((16,), jnp.float32)
    def _scat(j, _):
        def _inner(kk, _):
            idx = dstb[j, 0, pl.ds(kk * 16, 16)]
            plsc.addupdate_scatter(part, [idx], ones16)
            return 0
        lax.fori_loop(0, ECHUNK // 16, _inner, 0)
        return 0
    lax.fori_loop(0, NCHUNKS, _scat, 0)
    pltpu.sync_copy(part, parts_sh.at[sid, 0])
    plsc.subcore_barrier()
    pltpu.sync_copy(parts_sh.at[:, :, pl.ds(base, ROWS_PROC)], sumb)
    def _red(c, _):
        acc = jnp.ones((16,), jnp.float32)
        for p in range(NTILES):
            acc = acc + sumb[p, 0, pl.ds(c * 16, 16)]
        dinvb[pl.ds(c * 16, 16)] = _rsqrt16(acc)
        return 0
    lax.fori_loop(0, ROWS_PROC // 16, _red, 0)
    pltpu.sync_copy(dinvb, dinv_ref.at[pl.ds(base, ROWS_PROC)])


def _degree_dinv(dst4d):
    mesh = plsc.VectorSubcoreMesh(core_axis_name="c", subcore_axis_name="s")
    f = pl.kernel(
        _deg_body,
        out_type=jax.ShapeDtypeStruct((NPAD,), jnp.float32),
        mesh=mesh,
        compiler_params=pltpu.CompilerParams(needs_layout_passes=False, use_tc_tiling_on_sc=False),
        scratch_types=[
            pltpu.VMEM((NPAD,), jnp.float32),
            pltpu.VMEM((NCHUNKS, 1, ECHUNK), jnp.int32),
            pltpu.VMEM_SHARED((NTILES, 1, NPAD), jnp.float32),
            pltpu.VMEM((NTILES, 1, ROWS_PROC), jnp.float32),
            pltpu.VMEM((ROWS_PROC,), jnp.float32),
            pltpu.SemaphoreType.DMA,
        ],
    )
    return f(dst4d)


def _scale_rows(rowb, vecb):
    def _row(r, _):
        dv = plsc.load_gather(vecb, [jnp.full((16,), r, jnp.int32)])
        for j in range(FQ // 16):
            sl = pl.ds(j * 16, 16)
            rowb[r, sl] = rowb[r, sl] * dv
        return 0
    lax.fori_loop(0, ROWS_PROC, _row, 0)


def _hop_body(h_ref, dinv_ref, src_ref, dst_ref, u_ref,
              acc_sh, rowb, gbuf, srcb, dstb, dinvb, d2b, sem, ssem):
    cid = lax.axis_index("c")
    sid = lax.axis_index("s")
    base = sid * ROWS_PROC
    slab = pl.ds(base, ROWS_PROC)

    pltpu.sync_copy(src_ref.at[sid], srcb)
    pltpu.sync_copy(dst_ref.at[sid], dstb)
    pltpu.sync_copy(dinv_ref.at[slab], dinvb)
    def _sq(i, _):
        dv = dinvb[pl.ds(i * 16, 16)]
        d2b[pl.ds(i * 16, 16)] = dv * dv
        return 0
    lax.fori_loop(0, ROWS_PROC // 16, _sq, 0)

    for q in range(2):
        qq = 2 * cid + q
        @pl.when(sid < NTILES - 1)
        def _():
            pltpu.sync_copy(h_ref.at[qq, slab], rowb)
        @pl.when(sid == NTILES - 1)
        def _():
            pltpu.sync_copy(h_ref.at[qq, pl.ds(N - NTAIL, NTAIL)],
                            rowb.at[pl.ds(0, NTAIL)])
            zero16 = jnp.zeros((16,), jnp.float32)
            def _zp(r, _):
                for j in range(FQ // 16):
                    rowb[NTAIL + r, pl.ds(j * 16, 16)] = zero16
                return 0
            lax.fori_loop(0, ROWS_PROC - NTAIL, _zp, 0)
        _scale_rows(rowb, dinvb)
        pltpu.sync_copy(rowb, u_ref.at[qq, 0, slab])
        pltpu.sync_copy(rowb, acc_sh.at[slab])
        plsc.subcore_barrier()

        def _hop(k, _):
            src_view = u_ref.at[qq, k]
            for p in range(NBUF - 1):
                pltpu.async_copy(src_view.at[srcb.at[p, 0]], gbuf.at[p], sem)
            def _edge(j, _):
                b = lax.rem(j, NBUF)
                pltpu.make_async_copy(
                    src_view.at[srcb.at[j, 0]], gbuf.at[b], sem).wait()
                @pl.when(j < NCHUNKS - (NBUF - 1))
                def _():
                    pltpu.async_copy(
                        src_view.at[srcb.at[j + NBUF - 1, 0]],
                        gbuf.at[lax.rem(j + NBUF - 1, NBUF)], sem)
                pltpu.sync_copy(gbuf.at[b], acc_sh.at[dstb.at[j, 0]], add=True)
                return 0
            lax.fori_loop(0, NCHUNKS, _edge, 0)
            plsc.subcore_barrier()
            pltpu.sync_copy(acc_sh.at[slab], rowb)
            _scale_rows(rowb, d2b)
            pltpu.async_copy(rowb, u_ref.at[qq, k + 1, slab], sem)
            @pl.when(k < K - 1)
            def _():
                pltpu.async_copy(rowb, acc_sh.at[slab], ssem)
                pltpu.make_async_copy(rowb, acc_sh.at[slab], ssem).wait()
            pltpu.make_async_copy(rowb, u_ref.at[qq, k + 1, slab], sem).wait()
            plsc.subcore_barrier()
            return 0
        lax.fori_loop(0, K, _hop, 0)


def _khop(h_quarters, dinv, src4d, dst4d):
    mesh = plsc.VectorSubcoreMesh(core_axis_name="c", subcore_axis_name="s")
    f = pl.kernel(
        _hop_body,
        out_type=jax.ShapeDtypeStruct((4, K + 1, NPAD, FQ), jnp.float32),
        mesh=mesh,
        compiler_params=pltpu.CompilerParams(needs_layout_passes=False, use_tc_tiling_on_sc=False),
        scratch_types=[
            pltpu.VMEM_SHARED((NPAD, FQ), jnp.float32),
            pltpu.VMEM((ROWS_PROC, FQ), jnp.float32),
            pltpu.VMEM((NBUF, ECHUNK, FQ), jnp.float32),
            pltpu.VMEM((NCHUNKS, 1, ECHUNK), jnp.int32),
            pltpu.VMEM((NCHUNKS, 1, ECHUNK), jnp.int32),
            pltpu.VMEM((ROWS_PROC,), jnp.float32),
            pltpu.VMEM((ROWS_PROC,), jnp.float32),
            pltpu.SemaphoreType.DMA,
            pltpu.SemaphoreType.DMA,
        ],
    )
    return f(h_quarters, dinv, src4d, dst4d)


def _combine_body(h_ref, u_refq, dinv_ref, pw_ref, pb_ref, out_ref):
    pw = pw_ref[...]
    pb = pb_ref[0, 0]
    d12 = 1.0 / dinv_ref[...]
    hh = h_ref[...]
    uu = u_refq[...]
    acc = jnp.zeros_like(out_ref)
    for k in range(K + 1):
        if k == 0:
            pk = jnp.concatenate([hh[q] for q in range(4)], axis=-1)
        else:
            uk = jnp.concatenate([uu[q, k] for q in range(4)], axis=-1)
            pk = uk * d12
        s = jax.nn.sigmoid(jnp.sum(pk * pw, axis=1) + pb)
        acc = acc + s[:, None] * pk
    out_ref[...] = acc


def _combine(h_quarters, u_all, dinv, proj_w, proj_b):
    BN = 400
    grid = (N // BN,)
    return pl.pallas_call(
        _combine_body,
        grid=grid,
        in_specs=[
            pl.BlockSpec((4, BN, FQ), lambda i: (0, i, 0)),
            pl.BlockSpec((4, K + 1, BN, FQ), lambda i: (0, 0, i, 0)),
            pl.BlockSpec((BN, 1), lambda i: (i, 0)),
            pl.BlockSpec((1, OUT), lambda i: (0, 0)),
            pl.BlockSpec((1, 1), lambda i: (0, 0)),
        ],
        out_specs=pl.BlockSpec((BN, OUT), lambda i: (i, 0)),
        out_shape=jax.ShapeDtypeStruct((N, OUT), jnp.float32),
    )(h_quarters, u_all, dinv.reshape(NPAD, 1), proj_w.reshape(1, OUT),
      jnp.asarray(proj_b, jnp.float32).reshape(1, 1))


@jax.jit
def kernel(x, edge_index, W1, b1, W2, b2, proj_w, proj_b):
    pad = jnp.full((2, NTILES, EPT_PAD - EDGES_PER_TILE), NPAD - 1, jnp.int32)
    ei = jnp.concatenate(
        [edge_index.reshape(2, NTILES, EDGES_PER_TILE), pad], axis=2)
    src4d = ei[0].reshape(NTILES, NCHUNKS, 1, ECHUNK)
    dst4d = ei[1].reshape(NTILES, NCHUNKS, 1, ECHUNK)
    h_quarters = _mlp(x, W1, b1, W2, b2)
    dinv = _degree_dinv(dst4d)
    u_all = _khop(h_quarters, dinv, src4d, dst4d)
    return _combine(h_quarters, u_all, dinv, proj_w, proj_b)


# speedup vs baseline: 1.0714x; 1.0438x over previous
import jax
import jax.numpy as jnp
from jax import lax
from jax.experimental import pallas as pl
from jax.experimental.pallas import tpu as pltpu
from jax.experimental.pallas import tpu_sc as plsc

N = 10000
E = 160000
IN = 512
HID = 512
OUT = 256
K = 10
FQ = OUT // 4

NTILES = 16
NPAD = 10240
ROWS_PROC = 640
NTAIL = N - 15 * ROWS_PROC
EDGES_PER_TILE = E // NTILES
ECHUNK = 128
NCHUNKS = 79
EPT_PAD = NCHUNKS * ECHUNK
NBUF = 5
HROWS = ROWS_PROC // 2


def _mlp_body(x_ref, w1_ref, b1_ref, w2_ref, b2_ref, out_ref):
    h1 = jnp.maximum(
        jnp.dot(x_ref[...], w1_ref[...], preferred_element_type=jnp.float32)
        + b1_ref[...], 0.0)
    h2 = (jnp.dot(h1, w2_ref[...], preferred_element_type=jnp.float32)
          + b2_ref[...])
    for q in range(4):
        out_ref[q] = h2[:, q * FQ:(q + 1) * FQ]


def _mlp(x, W1, b1, W2, b2):
    BM = 400
    grid = (N // BM,)
    return pl.pallas_call(
        _mlp_body,
        grid=grid,
        in_specs=[
            pl.BlockSpec((BM, IN), lambda i: (i, 0)),
            pl.BlockSpec((IN, HID), lambda i: (0, 0)),
            pl.BlockSpec((1, HID), lambda i: (0, 0)),
            pl.BlockSpec((HID, OUT), lambda i: (0, 0)),
            pl.BlockSpec((1, OUT), lambda i: (0, 0)),
        ],
        out_specs=pl.BlockSpec((4, BM, FQ), lambda i: (0, i, 0)),
        out_shape=jax.ShapeDtypeStruct((4, N, FQ), jnp.float32),
    )(x, W1, b1.reshape(1, HID), W2, b2.reshape(1, OUT))


def _rsqrt16(d):
    i = plsc.bitcast(d, jnp.int32)
    i = jnp.int32(0x5F3759DF) - lax.shift_right_arithmetic(i, 1)
    y = plsc.bitcast(i, jnp.float32)
    for _ in range(4):
        y = y * (1.5 - 0.5 * d * y * y)
    return y


def _deg_body(dst_ref, dinv_ref, part, dstb, parts_sh, sumb, dinvb, sem):
    sid = lax.axis_index("s")
    base = sid * ROWS_PROC
    zero16 = jnp.zeros((16,), jnp.float32)
    def _z(i, _):
        part[pl.ds(i * 16, 16)] = zero16
        return 0
    lax.fori_loop(0, NPAD // 16, _z, 0)
    pltpu.sync_copy(dst_ref.at[sid], dstb)
    ones16 = jnp.ones((16,), jnp.float32)
    def _scat(j, _):
        def _inner(kk, _):
            idx = dstb[j, 0, pl.ds(kk * 16, 16)]
            plsc.addupdate_scatter(part, [idx], ones16)
            return 0
        lax.fori_loop(0, ECHUNK // 16, _inner, 0)
        return 0
    lax.fori_loop(0, NCHUNKS, _scat, 0)
    pltpu.sync_copy(part, parts_sh.at[sid, 0])
    plsc.subcore_barrier()
    pltpu.sync_copy(parts_sh.at[:, :, pl.ds(base, ROWS_PROC)], sumb)
    def _red(c, _):
        acc = jnp.ones((16,), jnp.float32)
        for p in range(NTILES):
            acc = acc + sumb[p, 0, pl.ds(c * 16, 16)]
        dinvb[pl.ds(c * 16, 16)] = _rsqrt16(acc)
        return 0
    lax.fori_loop(0, ROWS_PROC // 16, _red, 0)
    pltpu.sync_copy(dinvb, dinv_ref.at[pl.ds(base, ROWS_PROC)])


def _degree_dinv(dst4d):
    mesh = plsc.VectorSubcoreMesh(core_axis_name="c", subcore_axis_name="s")
    f = pl.kernel(
        _deg_body,
        out_type=jax.ShapeDtypeStruct((NPAD,), jnp.float32),
        mesh=mesh,
        compiler_params=pltpu.CompilerParams(needs_layout_passes=False, use_tc_tiling_on_sc=False),
        scratch_types=[
            pltpu.VMEM((NPAD,), jnp.float32),
            pltpu.VMEM((NCHUNKS, 1, ECHUNK), jnp.int32),
            pltpu.VMEM_SHARED((NTILES, 1, NPAD), jnp.float32),
            pltpu.VMEM((NTILES, 1, ROWS_PROC), jnp.float32),
            pltpu.VMEM((ROWS_PROC,), jnp.float32),
            pltpu.SemaphoreType.DMA,
        ],
    )
    return f(dst4d)


def _scale_rows(rowb, vecb, off):
    def _row(r, _):
        dv = plsc.load_gather(vecb, [jnp.full((16,), off + r, jnp.int32)])
        for j in range(FQ // 16):
            sl = pl.ds(j * 16, 16)
            rowb[r, sl] = rowb[r, sl] * dv
        return 0
    lax.fori_loop(0, HROWS, _row, 0)


def _hop_body(h_ref, dinv_ref, src_ref, dst_ref, u_ref,
              acc_sh, rowb, gbuf, srcb, dstb, dinvb, d2b, sem, ssem):
    cid = lax.axis_index("c")
    sid = lax.axis_index("s")
    base = sid * ROWS_PROC
    slab = pl.ds(base, ROWS_PROC)

    pltpu.sync_copy(src_ref.at[sid], srcb)
    pltpu.sync_copy(dst_ref.at[sid], dstb)
    pltpu.sync_copy(dinv_ref.at[slab], dinvb)
    def _sq(i, _):
        dv = dinvb[pl.ds(i * 16, 16)]
        d2b[pl.ds(i * 16, 16)] = dv * dv
        return 0
    lax.fori_loop(0, ROWS_PROC // 16, _sq, 0)

    for q in range(2):
        qq = 2 * cid + q
        for half in range(2):
            hsl = pl.ds(base + half * HROWS, HROWS)
            if half == 0:
                pltpu.sync_copy(h_ref.at[qq, hsl], rowb)
            else:
                nreal2 = NTAIL - HROWS
                @pl.when(sid < NTILES - 1)
                def _():
                    pltpu.sync_copy(h_ref.at[qq, hsl], rowb)
                @pl.when(sid == NTILES - 1)
                def _():
                    pltpu.sync_copy(h_ref.at[qq, pl.ds(N - nreal2, nreal2)],
                                    rowb.at[pl.ds(0, nreal2)])
                    zero16 = jnp.zeros((16,), jnp.float32)
                    def _zp(r, _):
                        for j in range(FQ // 16):
                            rowb[nreal2 + r, pl.ds(j * 16, 16)] = zero16
                        return 0
                    lax.fori_loop(0, HROWS - nreal2, _zp, 0)
            _scale_rows(rowb, dinvb, half * HROWS)
            pltpu.async_copy(rowb, u_ref.at[qq, 0, hsl], sem)
            pltpu.async_copy(rowb, acc_sh.at[hsl], ssem)
            pltpu.make_async_copy(rowb, acc_sh.at[hsl], ssem).wait()
            pltpu.make_async_copy(rowb, u_ref.at[qq, 0, hsl], sem).wait()
        plsc.subcore_barrier()

        def _hop(k, _):
            src_view = u_ref.at[qq, k]
            for p in range(NBUF - 1):
                pltpu.async_copy(src_view.at[srcb.at[p, 0]], gbuf.at[p], sem)
            def _edge(j, _):
                b = lax.rem(j, NBUF)
                pltpu.make_async_copy(
                    src_view.at[srcb.at[j, 0]], gbuf.at[b], sem).wait()
                @pl.when(j < NCHUNKS - (NBUF - 1))
                def _():
                    pltpu.async_copy(
                        src_view.at[srcb.at[j + NBUF - 1, 0]],
                        gbuf.at[lax.rem(j + NBUF - 1, NBUF)], sem)
                pltpu.sync_copy(gbuf.at[b], acc_sh.at[dstb.at[j, 0]], add=True)
                return 0
            lax.fori_loop(0, NCHUNKS, _edge, 0)
            plsc.subcore_barrier()
            for half in range(2):
                hsl = pl.ds(base + half * HROWS, HROWS)
                pltpu.sync_copy(acc_sh.at[hsl], rowb)
                _scale_rows(rowb, d2b, half * HROWS)
                pltpu.async_copy(rowb, u_ref.at[qq, k + 1, hsl], sem)
                @pl.when(k < K - 1)
                def _():
                    pltpu.async_copy(rowb, acc_sh.at[hsl], ssem)
                    pltpu.make_async_copy(rowb, acc_sh.at[hsl], ssem).wait()
                pltpu.make_async_copy(
                    rowb, u_ref.at[qq, k + 1, hsl], sem).wait()
            plsc.subcore_barrier()
            return 0
        lax.fori_loop(0, K, _hop, 0)


def _khop(h_quarters, dinv, src4d, dst4d):
    mesh = plsc.VectorSubcoreMesh(core_axis_name="c", subcore_axis_name="s")
    f = pl.kernel(
        _hop_body,
        out_type=jax.ShapeDtypeStruct((4, K + 1, NPAD, FQ), jnp.float32),
        mesh=mesh,
        compiler_params=pltpu.CompilerParams(needs_layout_passes=False, use_tc_tiling_on_sc=False),
        scratch_types=[
            pltpu.VMEM_SHARED((NPAD, FQ), jnp.float32),
            pltpu.VMEM((HROWS, FQ), jnp.float32),
            pltpu.VMEM((NBUF, ECHUNK, FQ), jnp.float32),
            pltpu.VMEM((NCHUNKS, 1, ECHUNK), jnp.int32),
            pltpu.VMEM((NCHUNKS, 1, ECHUNK), jnp.int32),
            pltpu.VMEM((ROWS_PROC,), jnp.float32),
            pltpu.VMEM((ROWS_PROC,), jnp.float32),
            pltpu.SemaphoreType.DMA,
            pltpu.SemaphoreType.DMA,
        ],
    )
    return f(h_quarters, dinv, src4d, dst4d)


def _combine_body(h_ref, u_refq, dinv_ref, pw_ref, pb_ref, out_ref):
    pw = pw_ref[...]
    pb = pb_ref[0, 0]
    d12 = 1.0 / dinv_ref[...]
    hh = h_ref[...]
    uu = u_refq[...]
    acc = jnp.zeros_like(out_ref)
    for k in range(K + 1):
        if k == 0:
            pk = jnp.concatenate([hh[q] for q in range(4)], axis=-1)
        else:
            uk = jnp.concatenate([uu[q, k] for q in range(4)], axis=-1)
            pk = uk * d12
        s = jax.nn.sigmoid(jnp.sum(pk * pw, axis=1) + pb)
        acc = acc + s[:, None] * pk
    out_ref[...] = acc


def _combine(h_quarters, u_all, dinv, proj_w, proj_b):
    BN = 400
    grid = (N // BN,)
    return pl.pallas_call(
        _combine_body,
        grid=grid,
        in_specs=[
            pl.BlockSpec((4, BN, FQ), lambda i: (0, i, 0)),
            pl.BlockSpec((4, K + 1, BN, FQ), lambda i: (0, 0, i, 0)),
            pl.BlockSpec((BN, 1), lambda i: (i, 0)),
            pl.BlockSpec((1, OUT), lambda i: (0, 0)),
            pl.BlockSpec((1, 1), lambda i: (0, 0)),
        ],
        out_specs=pl.BlockSpec((BN, OUT), lambda i: (i, 0)),
        out_shape=jax.ShapeDtypeStruct((N, OUT), jnp.float32),
    )(h_quarters, u_all, dinv.reshape(NPAD, 1), proj_w.reshape(1, OUT),
      jnp.asarray(proj_b, jnp.float32).reshape(1, 1))


@jax.jit
def kernel(x, edge_index, W1, b1, W2, b2, proj_w, proj_b):
    pad = jnp.full((2, NTILES, EPT_PAD - EDGES_PER_TILE), NPAD - 1, jnp.int32)
    ei = jnp.concatenate(
        [edge_index.reshape(2, NTILES, EDGES_PER_TILE), pad], axis=2)
    src4d = ei[0].reshape(NTILES, NCHUNKS, 1, ECHUNK)
    dst4d = ei[1].reshape(NTILES, NCHUNKS, 1, ECHUNK)
    h_quarters = _mlp(x, W1, b1, W2, b2)
    dinv = _degree_dinv(dst4d)
    u_all = _khop(h_quarters, dinv, src4d, dst4d)
    return _combine(h_quarters, u_all, dinv, proj_w, proj_b)
